# baseline (speedup 1.0000x reference)
import jax
import jax.numpy as jnp
from jax import lax
from jax.experimental import pallas as pl
from jax.experimental.pallas import tpu as pltpu
from jax.experimental.pallas import tpu_sc as plsc

N = 50000
E = 800000
GC = 64
HGC = GC // 4
NS, LANES = 16, 16
TPT = 3128
ACCN = NS * TPT
CH = 1000
K = 2
EPT = E // NS
NCH = EPT // CH
QW = TPT // 4

_MESH = plsc.VectorSubcoreMesh(
    core_axis_name="c", subcore_axis_name="s", num_cores=1, num_subcores=NS)


def _sc_pipeline(dst_hbm, src_hbm, ebase, dstv, srcv, isem, do_chunk):
    def fire_idx(i, k):
        off = ebase + i * CH
        pltpu.async_copy(dst_hbm.at[pl.ds(off, CH)], dstv[k], isem[k])
        if srcv is not None:
            pltpu.async_copy(src_hbm.at[pl.ds(off, CH)], srcv[k], isem[k])

    def wait_idx(i, k):
        off = ebase + i * CH
        pltpu.make_async_copy(dst_hbm.at[pl.ds(off, CH)], dstv[k],
                              isem[k]).wait()
        if srcv is not None:
            pltpu.make_async_copy(src_hbm.at[pl.ds(off, CH)], srcv[k],
                                  isem[k]).wait()

    for k in range(K):
        fire_idx(k, k)

    def steady(o, carry):
        for k in range(K):
            i = K * o + k
            wait_idx(i, k)
            do_chunk(i, k)
            fire_idx(i + K, k)
        return carry

    lax.fori_loop(0, NCH // K - 1, steady, 0)
    for k in range(K):
        i = NCH - K + k
        wait_idx(i, k)
        do_chunk(i, k)


def _sc_deg_body(dst_hbm, zdeg_hbm, deg_out, dstv0, dstv1, ones_v, stage_v,
                 deg_sp, isem0, isem1):
    s = lax.axis_index("s")
    ebase = s * EPT
    dstv = [dstv0, dstv1]
    isem = [isem0, isem1]

    def ones_body(j, carry):
        ones_v[pl.ds(j * LANES, LANES)] = jnp.ones((LANES,), jnp.float32)
        return carry

    lax.fori_loop(0, CH // LANES, ones_body, 0)
    pltpu.sync_copy(zdeg_hbm, stage_v)
    pltpu.sync_copy(stage_v, deg_sp.at[pl.ds(s * TPT, TPT)])
    plsc.subcore_barrier()

    def do_chunk(i, k):
        pltpu.sync_copy(ones_v, deg_sp.at[dstv[k]], add=True)

    _sc_pipeline(dst_hbm, None, ebase, dstv, None, isem, do_chunk)

    plsc.subcore_barrier()
    pltpu.sync_copy(deg_sp.at[pl.ds(s * TPT, TPT)], stage_v)
    pltpu.sync_copy(stage_v, deg_out.at[s])


_sc_deg = pl.kernel(
    _sc_deg_body,
    out_type=jax.ShapeDtypeStruct((NS, TPT), jnp.float32),
    mesh=_MESH,
    scratch_types=[
        pltpu.VMEM((CH,), jnp.int32),
        pltpu.VMEM((CH,), jnp.int32),
        pltpu.VMEM((CH,), jnp.float32),
        pltpu.VMEM((TPT,), jnp.float32),
        pltpu.VMEM_SHARED((ACCN,), jnp.float32),
        pltpu.SemaphoreType.DMA,
        pltpu.SemaphoreType.DMA,
    ],
    compiler_params=pltpu.CompilerParams(use_tc_tiling_on_sc=False),
)


def _sc_scatter_body(src_hbm, dst_hbm, y0_hbm, y1_hbm, y2_hbm, y3_hbm,
                     zrows_hbm, out20, out21, out22, out23, dstv0, dstv1,
                     srcv0, srcv1, rows0, rows1, stage_v, acc_sp, isem0,
                     isem1, gsem0, gsem1):
    s = lax.axis_index("s")
    ebase = s * EPT
    dstv = [dstv0, dstv1]
    srcv = [srcv0, srcv1]
    rowsv = [rows0, rows1]
    isem = [isem0, isem1]
    gsem = [gsem0, gsem1]

    for y_hbm, out2 in ((y0_hbm, out20), (y1_hbm, out21),
                        (y2_hbm, out22), (y3_hbm, out23)):
        pltpu.sync_copy(zrows_hbm, stage_v)
        for q in range(4):
            pltpu.sync_copy(stage_v,
                            acc_sp.at[pl.ds(s * TPT + q * QW, QW), :])
        plsc.subcore_barrier()

        def do_chunk(i, k):
            pltpu.async_copy(y_hbm.at[srcv[k]], rowsv[k], gsem[k])
            pltpu.make_async_copy(y_hbm.at[srcv[k]], rowsv[k],
                                  gsem[k]).wait()
            pltpu.sync_copy(rowsv[k], acc_sp.at[dstv[k]], add=True)

        _sc_pipeline(dst_hbm, src_hbm, ebase, dstv, srcv, isem, do_chunk)

        plsc.subcore_barrier()
        for q in range(4):
            pltpu.sync_copy(acc_sp.at[pl.ds(s * TPT + q * QW, QW), :],
                            stage_v)
            pltpu.sync_copy(stage_v, out2.at[s, pl.ds(q * QW, QW), :])
        plsc.subcore_barrier()


_sc_scatter = pl.kernel(
    _sc_scatter_body,
    out_type=[jax.ShapeDtypeStruct((NS, TPT, HGC), jnp.float32)
              for _ in range(4)],
    mesh=_MESH,
    scratch_types=[
        pltpu.VMEM((CH,), jnp.int32),
        pltpu.VMEM((CH,), jnp.int32),
        pltpu.VMEM((CH,), jnp.int32),
        pltpu.VMEM((CH,), jnp.int32),
        pltpu.VMEM((CH, HGC), jnp.float32),
        pltpu.VMEM((CH, HGC), jnp.float32),
        pltpu.VMEM((QW, HGC), jnp.float32),
        pltpu.VMEM_SHARED((ACCN, HGC), jnp.float32),
        pltpu.SemaphoreType.DMA,
        pltpu.SemaphoreType.DMA,
        pltpu.SemaphoreType.DMA,
        pltpu.SemaphoreType.DMA,
    ],
    compiler_params=pltpu.CompilerParams(use_tc_tiling_on_sc=False),
)


def _dot_t(a, w):
    return lax.dot_general(a, w, (((1,), (1,)), ((), ())),
                           preferred_element_type=jnp.float32)


def _tc1_body(feats, W1, b1, W2, b2, Wgc, deg, nf_out, y0_out, y1_out,
              y2_out, y3_out):
    t1 = _dot_t(feats[...], W1[...]) + b1[...]
    nf = _dot_t(t1, W2[...]) + b2[...]
    xw = _dot_t(nf, Wgc[...])
    dinv = lax.rsqrt(deg[...] + 1.0)
    y = xw * dinv
    nf_out[...] = nf
    y0_out[...] = y[:, 0 * HGC:1 * HGC]
    y1_out[...] = y[:, 1 * HGC:2 * HGC]
    y2_out[...] = y[:, 2 * HGC:3 * HGC]
    y3_out[...] = y[:, 3 * HGC:4 * HGC]


def _tc2_body(nf, y0, y1, y2, y3, o20, o21, o22, o23, deg, Wc1, Wq0, Wq1,
              Wq2, Wq3, bc, bg0, bg1, bg2, bg3, out):
    dinv = lax.rsqrt(deg[...] + 1.0)
    acc = _dot_t(nf[...], Wc1[...]) + bc[...]
    for yq, oq, wq, bgq in ((y0, o20, Wq0, bg0), (y1, o21, Wq1, bg1),
                            (y2, o22, Wq2, bg2), (y3, o23, Wq3, bg3)):
        gcq = dinv * (oq[...] + yq[...]) + bgq[...]
        acc = acc + _dot_t(gcq, wq[...])
    out[...] = acc


def _row_spec(cols):
    return pl.BlockSpec((ROWB, cols), lambda i: (i, 0))


def _full_spec(r, c):
    return pl.BlockSpec((r, c), lambda i: (0, 0))


ROWB = 2000
_GRID = N // ROWB

_tc1 = pl.pallas_call(
    _tc1_body,
    grid=(_GRID,),
    in_specs=[
        _row_spec(128),
        _full_spec(128, 128), _full_spec(1, 128),
        _full_spec(64, 128), _full_spec(1, 64),
        _full_spec(64, 64),
        _row_spec(1),
    ],
    out_specs=[_row_spec(GC)] + [_row_spec(HGC) for _ in range(4)],
    out_shape=[jax.ShapeDtypeStruct((N, GC), jnp.float32)] +
    [jax.ShapeDtypeStruct((N, HGC), jnp.float32) for _ in range(4)],
    compiler_params=pltpu.CompilerParams(
        dimension_semantics=("arbitrary",)),
)

_tc2 = pl.pallas_call(
    _tc2_body,
    grid=(_GRID,),
    in_specs=[_row_spec(GC)] + [_row_spec(HGC) for _ in range(8)] +
    [_row_spec(1), _full_spec(128, 64)] +
    [_full_spec(128, HGC) for _ in range(4)] +
    [_full_spec(1, 128)] + [_full_spec(1, HGC) for _ in range(4)],
    out_specs=[_row_spec(128)],
    out_shape=[jax.ShapeDtypeStruct((N, 128), jnp.float32)],
    compiler_params=pltpu.CompilerParams(
        dimension_semantics=("arbitrary",)),
)


@jax.jit
def kernel(feats, edges, batch, W1, b1, W2, b2, Wgc, bgc, Wc, bc):
    src = edges[0]
    dst = edges[1]
    zdeg = jnp.zeros((TPT,), jnp.float32)
    zrows = jnp.zeros((QW, HGC), jnp.float32)

    deg = _sc_deg(dst, zdeg).reshape(ACCN)[:N].reshape(N, 1)

    nf, y0, y1, y2, y3 = _tc1(feats, W1, b1.reshape(1, -1), W2,
                              b2.reshape(1, -1), Wgc, deg)

    o2 = _sc_scatter(src, dst, y0, y1, y2, y3, zrows)
    o2 = [o.reshape(ACCN, HGC)[:N] for o in o2]

    wq = [Wc[:, GC + i * HGC:GC + (i + 1) * HGC] for i in range(4)]
    bg = [bgc[i * HGC:(i + 1) * HGC].reshape(1, -1) for i in range(4)]
    comb, = _tc2(nf, y0, y1, y2, y3, o2[0], o2[1], o2[2], o2[3], deg,
                 Wc[:, :GC], wq[0], wq[1], wq[2], wq[3],
                 bc.reshape(1, -1), bg[0], bg[1], bg[2], bg[3])
    return (comb, edges, batch)


# speedup vs baseline: 1.3439x; 1.3439x over previous
import jax
import jax.numpy as jnp
from jax import lax
from jax.experimental import pallas as pl
from jax.experimental.pallas import tpu as pltpu
from jax.experimental.pallas import tpu_sc as plsc

N = 50000
E = 800000
GC = 64
HGC = GC // 2
NC, NS, LANES = 2, 16, 16
HALF = N // NC
TPT = 1568
ACC = NS * TPT
CH = 400
K = 2
EPT = E // NS
NCH = EPT // CH
NBLK = (NCH - 1) // K
ROWB = 2000

_MESH = plsc.VectorSubcoreMesh(
    core_axis_name="c", subcore_axis_name="s", num_cores=NC, num_subcores=NS)


def _remap_dst(dst_v, base):
    iota16 = lax.iota(jnp.int32, 16)

    def body(j, carry):
        d = dst_v[pl.ds(j * LANES, LANES)]
        t = d - base
        ok = plsc.bitcast(t, jnp.uint32) < jnp.uint32(HALF)
        dst_v[pl.ds(j * LANES, LANES)] = jnp.where(ok, t, HALF + iota16)
        return carry

    lax.fori_loop(0, CH // LANES, body, 0)


def _sc_pipeline(src_hbm, dst_hbm, ebase, base, dstv, srcv, isem,
                 ssem, fire_body, wait_fire_body, tail_body):
    def fire_idx(b, q):
        off = ebase + b * (K * CH)
        for k in range(K):
            pltpu.async_copy(dst_hbm.at[pl.ds(off + k * CH, CH)], dstv[q][k],
                             isem[q])
            if srcv is not None:
                pltpu.async_copy(src_hbm.at[pl.ds(off + k * CH, CH)],
                                 srcv[q][k], isem[q])

    def wait_idx(b, q):
        off = ebase + b * (K * CH)
        for k in range(K):
            pltpu.make_async_copy(dst_hbm.at[pl.ds(off + k * CH, CH)],
                                  dstv[q][k], isem[q]).wait()
            if srcv is not None:
                pltpu.make_async_copy(src_hbm.at[pl.ds(off + k * CH, CH)],
                                      srcv[q][k], isem[q]).wait()

    def drain_scatters(q):
        for k in range(K):
            ssem[q][k].wait()

    def block(b, q, drain_prev, prefetch_next):
        wait_idx(b, q)
        for k in range(K):
            _remap_dst(dstv[q][k], base)
        for k in range(K):
            fire_body(q, k)
        if drain_prev:
            drain_scatters(1 - q)
        if prefetch_next:
            fire_idx(b + 1, 1 - q)
        for k in range(K):
            wait_fire_body(q, k)

    assert NBLK % 2 == 0
    fire_idx(0, 0)
    block(0, 0, drain_prev=False, prefetch_next=True)
    block(1, 1, drain_prev=True, prefetch_next=True)

    def steady(i2, carry):
        b = 2 + 2 * i2
        block(b, 0, drain_prev=True, prefetch_next=True)
        block(b + 1, 1, drain_prev=True, prefetch_next=True)
        return carry

    lax.fori_loop(0, (NBLK - 4) // 2, steady, 0)
    block(NBLK - 2, 0, drain_prev=True, prefetch_next=True)
    block(NBLK - 1, 1, drain_prev=True, prefetch_next=False)
    drain_scatters(1)
    tail_body()


def _sc_deg_body(dst_hbm, zdeg_hbm, deg_out, *refs):
    it = iter(refs)
    dstv = [[next(it) for _ in range(K)] for _ in range(2)]
    ones_v = next(it)
    stage_v = next(it)
    deg_sp = next(it)
    isem = [next(it) for _ in range(2)]
    ssem = [[next(it) for _ in range(K)] for _ in range(2)]

    c = lax.axis_index("c")
    s = lax.axis_index("s")
    base = c * HALF
    ebase = s * EPT

    def ones_body(j, carry):
        ones_v[pl.ds(j * LANES, LANES)] = jnp.ones((LANES,), jnp.float32)
        return carry

    lax.fori_loop(0, CH // LANES, ones_body, 0)
    pltpu.sync_copy(zdeg_hbm, stage_v)
    pltpu.sync_copy(stage_v, deg_sp.at[pl.ds(s * TPT, TPT)])
    plsc.subcore_barrier()

    def fire_body(q, k):
        pass

    def wait_fire_body(q, k):
        pltpu.async_copy(ones_v, deg_sp.at[dstv[q][k]], ssem[q][k], add=True)

    def tail_body():
        off = ebase + (NCH - 1) * CH
        pltpu.sync_copy(dst_hbm.at[pl.ds(off, CH)], dstv[1][0])
        _remap_dst(dstv[1][0], base)
        pltpu.sync_copy(ones_v, deg_sp.at[dstv[1][0]], add=True)

    class _SemWrap:
        def __init__(self, sem, src, dst):
            self._sem, self._src, self._dst = sem, src, dst

        def wait(self):
            pltpu.make_async_copy(self._src, self._dst, self._sem).wait()

    wsem = [[_SemWrap(ssem[q][k], ones_v, deg_sp.at[dstv[q][k]])
             for k in range(K)] for q in range(2)]

    _sc_pipeline(None, dst_hbm, ebase, base, dstv, None, isem, wsem,
                 fire_body, wait_fire_body, tail_body)

    plsc.subcore_barrier()
    pltpu.sync_copy(deg_sp.at[pl.ds(s * TPT, TPT)], stage_v)
    pltpu.sync_copy(stage_v, deg_out.at[c, s])


_sc_deg = pl.kernel(
    _sc_deg_body,
    out_type=jax.ShapeDtypeStruct((NC, NS, TPT), jnp.float32),
    mesh=_MESH,
    scratch_types=(
        [pltpu.VMEM((CH,), jnp.int32) for _ in range(2 * K)] +
        [pltpu.VMEM((CH,), jnp.float32),
         pltpu.VMEM((TPT,), jnp.float32),
         pltpu.VMEM_SHARED((ACC,), jnp.float32)] +
        [pltpu.SemaphoreType.DMA for _ in range(2 + 2 * K)]
    ),
    compiler_params=pltpu.CompilerParams(use_tc_tiling_on_sc=False),
)


def _sc_scatter_body(src_hbm, dst_hbm, ya_hbm, yb_hbm, zrows_hbm, out2a,
                     out2b, *refs):
    it = iter(refs)
    dstv = [[next(it) for _ in range(K)] for _ in range(2)]
    srcv = [[next(it) for _ in range(K)] for _ in range(2)]
    rowsv = [[next(it) for _ in range(K)] for _ in range(2)]
    stage_v = next(it)
    acc_sp = next(it)
    isem = [next(it) for _ in range(2)]
    gsem = [[next(it) for _ in range(K)] for _ in range(2)]
    ssem = [[next(it) for _ in range(K)] for _ in range(2)]

    c = lax.axis_index("c")
    s = lax.axis_index("s")
    base = c * HALF
    ebase = s * EPT
    Q4 = TPT // 4

    class _SemWrap:
        def __init__(self, sem, src, dst):
            self._sem, self._src, self._dst = sem, src, dst

        def wait(self):
            pltpu.make_async_copy(self._src, self._dst, self._sem).wait()

    for y_hbm, out2 in ((ya_hbm, out2a), (yb_hbm, out2b)):
        for q in range(4):
            pltpu.sync_copy(zrows_hbm.at[pl.ds(q * Q4, Q4), :], stage_v)
            pltpu.sync_copy(
                stage_v,
                acc_sp.at[pl.ds(s * TPT + q * Q4, Q4), :])
        plsc.subcore_barrier()

        def fire_body(q, k):
            pltpu.async_copy(y_hbm.at[srcv[q][k]], rowsv[q][k], gsem[q][k])

        def wait_fire_body(q, k):
            pltpu.make_async_copy(y_hbm.at[srcv[q][k]], rowsv[q][k],
                                  gsem[q][k]).wait()
            pltpu.async_copy(rowsv[q][k], acc_sp.at[dstv[q][k]], ssem[q][k],
                             add=True)

        def tail_body():
            off = ebase + (NCH - 1) * CH
            pltpu.sync_copy(dst_hbm.at[pl.ds(off, CH)], dstv[1][0])
            pltpu.sync_copy(src_hbm.at[pl.ds(off, CH)], srcv[1][0])
            _remap_dst(dstv[1][0], base)
            pltpu.sync_copy(y_hbm.at[srcv[1][0]], rowsv[1][0])
            pltpu.sync_copy(rowsv[1][0], acc_sp.at[dstv[1][0]], add=True)

        wsem = [[_SemWrap(ssem[q][k], rowsv[q][k], acc_sp.at[dstv[q][k]])
                 for k in range(K)] for q in range(2)]

        _sc_pipeline(src_hbm, dst_hbm, ebase, base, dstv, srcv, isem, wsem,
                     fire_body, wait_fire_body, tail_body)

        plsc.subcore_barrier()
        for q in range(4):
            pltpu.sync_copy(
                acc_sp.at[pl.ds(s * TPT + q * Q4, Q4), :],
                stage_v)
            pltpu.sync_copy(stage_v, out2.at[c, s, pl.ds(q * Q4, Q4), :])
        plsc.subcore_barrier()


_sc_scatter = pl.kernel(
    _sc_scatter_body,
    out_type=[
        jax.ShapeDtypeStruct((NC, NS, TPT, HGC), jnp.float32),
        jax.ShapeDtypeStruct((NC, NS, TPT, HGC), jnp.float32),
    ],
    mesh=_MESH,
    scratch_types=(
        [pltpu.VMEM((CH,), jnp.int32) for _ in range(2 * K)] +
        [pltpu.VMEM((CH,), jnp.int32) for _ in range(2 * K)] +
        [pltpu.VMEM((CH, HGC), jnp.float32) for _ in range(2 * K)] +
        [pltpu.VMEM((TPT // 4, HGC), jnp.float32),
         pltpu.VMEM_SHARED((ACC, HGC), jnp.float32)] +
        [pltpu.SemaphoreType.DMA for _ in range(2 + 4 * K)]
    ),
    compiler_params=pltpu.CompilerParams(use_tc_tiling_on_sc=False),
)


def _dot_t(a, w):
    return lax.dot_general(a, w, (((1,), (1,)), ((), ())),
                           preferred_element_type=jnp.float32)


def _tc1_body(feats, W1, b1, W2, b2, Wgc, deg, nf_out, y_out):
    t1 = _dot_t(feats[...], W1[...]) + b1[...]
    nf = _dot_t(t1, W2[...]) + b2[...]
    xw = _dot_t(nf, Wgc[...])
    dinv = lax.rsqrt(deg[...] + 1.0)
    nf_out[...] = nf
    y_out[...] = xw * dinv


def _tc2_body(nf, y, out2, deg, Wc1, Wc2, bc, bgc, out):
    dinv = lax.rsqrt(deg[...] + 1.0)
    gc = dinv * (out2[...] + y[...]) + bgc[...]
    out[...] = _dot_t(nf[...], Wc1[...]) + _dot_t(gc, Wc2[...]) + bc[...]


def _row_spec(cols):
    return pl.BlockSpec((ROWB, cols), lambda i: (i, 0))


def _full_spec(r, c):
    return pl.BlockSpec((r, c), lambda i: (0, 0))


_GRID = N // ROWB

_tc1 = pl.pallas_call(
    _tc1_body,
    grid=(_GRID,),
    in_specs=[
        _row_spec(128),
        _full_spec(128, 128), _full_spec(1, 128),
        _full_spec(64, 128), _full_spec(1, 64),
        _full_spec(64, 64),
        _row_spec(1),
    ],
    out_specs=[_row_spec(GC), _row_spec(GC)],
    out_shape=[
        jax.ShapeDtypeStruct((N, GC), jnp.float32),
        jax.ShapeDtypeStruct((N, GC), jnp.float32),
    ],
    compiler_params=pltpu.CompilerParams(
        dimension_semantics=("arbitrary",)),
)

_tc2 = pl.pallas_call(
    _tc2_body,
    grid=(_GRID,),
    in_specs=[
        _row_spec(GC), _row_spec(GC), _row_spec(GC), _row_spec(1),
        _full_spec(128, 64), _full_spec(128, 64),
        _full_spec(1, 128), _full_spec(1, 64),
    ],
    out_specs=[_row_spec(128)],
    out_shape=[jax.ShapeDtypeStruct((N, 128), jnp.float32)],
    compiler_params=pltpu.CompilerParams(
        dimension_semantics=("arbitrary",)),
)


@jax.jit
def kernel(feats, edges, batch, W1, b1, W2, b2, Wgc, bgc, Wc, bc):
    src = edges[0]
    dst = edges[1]
    zdeg = jnp.zeros((TPT,), jnp.float32)
    zrows = jnp.zeros((TPT, HGC), jnp.float32)

    deg_raw = _sc_deg(dst, zdeg)
    deg = jnp.concatenate([
        deg_raw[0].reshape(ACC)[:HALF],
        deg_raw[1].reshape(ACC)[:HALF],
    ]).reshape(N, 1)

    nf, y = _tc1(feats, W1, b1.reshape(1, -1), W2, b2.reshape(1, -1), Wgc,
                 deg)

    out2a_raw, out2b_raw = _sc_scatter(src, dst, y[:, :HGC], y[:, HGC:],
                                       zrows)
    out2 = jnp.concatenate([
        jnp.concatenate([out2a_raw[i].reshape(ACC, HGC)[:HALF],
                         out2b_raw[i].reshape(ACC, HGC)[:HALF]], axis=1)
        for i in range(NC)
    ], axis=0)

    comb, = _tc2(nf, y, out2, deg, Wc[:, :GC], Wc[:, GC:],
                 bc.reshape(1, -1), bgc.reshape(1, -1))
    return (comb, edges, batch)


# speedup vs baseline: 1.3557x; 1.0087x over previous
import jax
import jax.numpy as jnp
from jax import lax
from jax.experimental import pallas as pl
from jax.experimental.pallas import tpu as pltpu
from jax.experimental.pallas import tpu_sc as plsc

N = 50000
E = 800000
GC = 64
HGC = GC // 2
NC, NS, LANES = 2, 16, 16
HALF = N // NC
TPT = 1568
ACC = NS * TPT
CH = 400
K = 2
EPT = E // NS
NCH = EPT // CH
NBLK = (NCH - 1) // K
DCH = 2000
DNCH = EPT // DCH
DNBLK = (DNCH - 1) // K
ROWB = 2000

_MESH = plsc.VectorSubcoreMesh(
    core_axis_name="c", subcore_axis_name="s", num_cores=NC, num_subcores=NS)


def _remap_dst(dst_v, base, ch):
    iota16 = lax.iota(jnp.int32, 16)

    def body(j, carry):
        d = dst_v[pl.ds(j * LANES, LANES)]
        t = d - base
        ok = plsc.bitcast(t, jnp.uint32) < jnp.uint32(HALF)
        dst_v[pl.ds(j * LANES, LANES)] = jnp.where(ok, t, HALF + iota16)
        return carry

    lax.fori_loop(0, ch // LANES, body, 0)


def _sc_pipeline(src_hbm, dst_hbm, ebase, base, dstv, srcv, isem,
                 ssem, fire_body, wait_fire_body, tail_body, ch, nblk):
    def fire_idx(b, q):
        off = ebase + b * (K * ch)
        for k in range(K):
            pltpu.async_copy(dst_hbm.at[pl.ds(off + k * ch, ch)], dstv[q][k],
                             isem[q])
            if srcv is not None:
                pltpu.async_copy(src_hbm.at[pl.ds(off + k * ch, ch)],
                                 srcv[q][k], isem[q])

    def wait_idx(b, q):
        off = ebase + b * (K * ch)
        for k in range(K):
            pltpu.make_async_copy(dst_hbm.at[pl.ds(off + k * ch, ch)],
                                  dstv[q][k], isem[q]).wait()
            if srcv is not None:
                pltpu.make_async_copy(src_hbm.at[pl.ds(off + k * ch, ch)],
                                      srcv[q][k], isem[q]).wait()

    def drain_scatters(q):
        for k in range(K):
            ssem[q][k].wait()

    def block(b, q, drain_prev, prefetch_next):
        wait_idx(b, q)
        for k in range(K):
            _remap_dst(dstv[q][k], base, ch)
        for k in range(K):
            fire_body(q, k)
        if drain_prev:
            drain_scatters(1 - q)
        if prefetch_next:
            fire_idx(b + 1, 1 - q)
        for k in range(K):
            wait_fire_body(q, k)

    assert nblk % 2 == 0
    fire_idx(0, 0)
    block(0, 0, drain_prev=False, prefetch_next=True)
    block(1, 1, drain_prev=True, prefetch_next=True)

    def steady(i2, carry):
        b = 2 + 2 * i2
        block(b, 0, drain_prev=True, prefetch_next=True)
        block(b + 1, 1, drain_prev=True, prefetch_next=True)
        return carry

    lax.fori_loop(0, (nblk - 4) // 2, steady, 0)
    block(nblk - 2, 0, drain_prev=True, prefetch_next=True)
    block(nblk - 1, 1, drain_prev=True, prefetch_next=False)
    drain_scatters(1)
    tail_body()


def _sc_deg_body(dst_hbm, zdeg_hbm, deg_out, *refs):
    it = iter(refs)
    dstv = [[next(it) for _ in range(K)] for _ in range(2)]
    ones_v = next(it)
    stage_v = next(it)
    deg_sp = next(it)
    isem = [next(it) for _ in range(2)]
    ssem = [[next(it) for _ in range(K)] for _ in range(2)]

    c = lax.axis_index("c")
    s = lax.axis_index("s")
    base = c * HALF
    ebase = s * EPT

    def ones_body(j, carry):
        ones_v[pl.ds(j * LANES, LANES)] = jnp.ones((LANES,), jnp.float32)
        return carry

    lax.fori_loop(0, DCH // LANES, ones_body, 0)
    pltpu.sync_copy(zdeg_hbm, stage_v)
    pltpu.sync_copy(stage_v, deg_sp.at[pl.ds(s * TPT, TPT)])
    plsc.subcore_barrier()

    def fire_body(q, k):
        pass

    def wait_fire_body(q, k):
        pltpu.async_copy(ones_v, deg_sp.at[dstv[q][k]], ssem[q][k], add=True)

    def tail_body():
        off = ebase + (DNCH - 1) * DCH
        pltpu.sync_copy(dst_hbm.at[pl.ds(off, DCH)], dstv[1][0])
        _remap_dst(dstv[1][0], base, DCH)
        pltpu.sync_copy(ones_v, deg_sp.at[dstv[1][0]], add=True)

    class _SemWrap:
        def __init__(self, sem, src, dst):
            self._sem, self._src, self._dst = sem, src, dst

        def wait(self):
            pltpu.make_async_copy(self._src, self._dst, self._sem).wait()

    wsem = [[_SemWrap(ssem[q][k], ones_v, deg_sp.at[dstv[q][k]])
             for k in range(K)] for q in range(2)]

    _sc_pipeline(None, dst_hbm, ebase, base, dstv, None, isem, wsem,
                 fire_body, wait_fire_body, tail_body, DCH, DNBLK)

    plsc.subcore_barrier()
    pltpu.sync_copy(deg_sp.at[pl.ds(s * TPT, TPT)], stage_v)
    pltpu.sync_copy(stage_v, deg_out.at[c, s])


_sc_deg = pl.kernel(
    _sc_deg_body,
    out_type=jax.ShapeDtypeStruct((NC, NS, TPT), jnp.float32),
    mesh=_MESH,
    scratch_types=(
        [pltpu.VMEM((DCH,), jnp.int32) for _ in range(2 * K)] +
        [pltpu.VMEM((DCH,), jnp.float32),
         pltpu.VMEM((TPT,), jnp.float32),
         pltpu.VMEM_SHARED((ACC,), jnp.float32)] +
        [pltpu.SemaphoreType.DMA for _ in range(2 + 2 * K)]
    ),
    compiler_params=pltpu.CompilerParams(use_tc_tiling_on_sc=False),
)


def _sc_scatter_body(src_hbm, dst_hbm, ya_hbm, yb_hbm, zrows_hbm, out2a,
                     out2b, *refs):
    it = iter(refs)
    dstv = [[next(it) for _ in range(K)] for _ in range(2)]
    srcv = [[next(it) for _ in range(K)] for _ in range(2)]
    rowsv = [[next(it) for _ in range(K)] for _ in range(2)]
    stage_v = next(it)
    acc_sp = next(it)
    isem = [next(it) for _ in range(2)]
    gsem = [[next(it) for _ in range(K)] for _ in range(2)]
    ssem = [[next(it) for _ in range(K)] for _ in range(2)]

    c = lax.axis_index("c")
    s = lax.axis_index("s")
    base = c * HALF
    ebase = s * EPT
    Q4 = TPT // 4

    class _SemWrap:
        def __init__(self, sem, src, dst):
            self._sem, self._src, self._dst = sem, src, dst

        def wait(self):
            pltpu.make_async_copy(self._src, self._dst, self._sem).wait()

    for y_hbm, out2 in ((ya_hbm, out2a), (yb_hbm, out2b)):
        for q in range(4):
            pltpu.sync_copy(zrows_hbm.at[pl.ds(q * Q4, Q4), :], stage_v)
            pltpu.sync_copy(
                stage_v,
                acc_sp.at[pl.ds(s * TPT + q * Q4, Q4), :])
        plsc.subcore_barrier()

        def fire_body(q, k):
            pltpu.async_copy(y_hbm.at[srcv[q][k]], rowsv[q][k], gsem[q][k])

        def wait_fire_body(q, k):
            pltpu.make_async_copy(y_hbm.at[srcv[q][k]], rowsv[q][k],
                                  gsem[q][k]).wait()
            pltpu.async_copy(rowsv[q][k], acc_sp.at[dstv[q][k]], ssem[q][k],
                             add=True)

        def tail_body():
            off = ebase + (NCH - 1) * CH
            pltpu.sync_copy(dst_hbm.at[pl.ds(off, CH)], dstv[1][0])
            pltpu.sync_copy(src_hbm.at[pl.ds(off, CH)], srcv[1][0])
            _remap_dst(dstv[1][0], base, CH)
            pltpu.sync_copy(y_hbm.at[srcv[1][0]], rowsv[1][0])
            pltpu.sync_copy(rowsv[1][0], acc_sp.at[dstv[1][0]], add=True)

        wsem = [[_SemWrap(ssem[q][k], rowsv[q][k], acc_sp.at[dstv[q][k]])
                 for k in range(K)] for q in range(2)]

        _sc_pipeline(src_hbm, dst_hbm, ebase, base, dstv, srcv, isem, wsem,
                     fire_body, wait_fire_body, tail_body, CH, NBLK)

        plsc.subcore_barrier()
        for q in range(4):
            pltpu.sync_copy(
                acc_sp.at[pl.ds(s * TPT + q * Q4, Q4), :],
                stage_v)
            pltpu.sync_copy(stage_v, out2.at[c, s, pl.ds(q * Q4, Q4), :])
        plsc.subcore_barrier()


_sc_scatter = pl.kernel(
    _sc_scatter_body,
    out_type=[
        jax.ShapeDtypeStruct((NC, NS, TPT, HGC), jnp.float32),
        jax.ShapeDtypeStruct((NC, NS, TPT, HGC), jnp.float32),
    ],
    mesh=_MESH,
    scratch_types=(
        [pltpu.VMEM((CH,), jnp.int32) for _ in range(2 * K)] +
        [pltpu.VMEM((CH,), jnp.int32) for _ in range(2 * K)] +
        [pltpu.VMEM((CH, HGC), jnp.float32) for _ in range(2 * K)] +
        [pltpu.VMEM((TPT // 4, HGC), jnp.float32),
         pltpu.VMEM_SHARED((ACC, HGC), jnp.float32)] +
        [pltpu.SemaphoreType.DMA for _ in range(2 + 4 * K)]
    ),
    compiler_params=pltpu.CompilerParams(use_tc_tiling_on_sc=False),
)


def _dot_t(a, w):
    return lax.dot_general(a, w, (((1,), (1,)), ((), ())),
                           preferred_element_type=jnp.float32)


def _tc1a_body(feats, W1, b1, W2, b2, Wgc, nf_out, xw_out):
    t1 = _dot_t(feats[...], W1[...]) + b1[...]
    nf = _dot_t(t1, W2[...]) + b2[...]
    nf_out[...] = nf
    xw_out[...] = _dot_t(nf, Wgc[...])


def _tc1b_body(xw, deg, y_out):
    y_out[...] = xw[...] * lax.rsqrt(deg[...] + 1.0)


def _tc2_body(nf, y, out2, deg, Wc1, Wc2, bc, bgc, out):
    dinv = lax.rsqrt(deg[...] + 1.0)
    gc = dinv * (out2[...] + y[...]) + bgc[...]
    out[...] = _dot_t(nf[...], Wc1[...]) + _dot_t(gc, Wc2[...]) + bc[...]


def _row_spec(cols):
    return pl.BlockSpec((ROWB, cols), lambda i: (i, 0))


def _full_spec(r, c):
    return pl.BlockSpec((r, c), lambda i: (0, 0))


_GRID = N // ROWB

_tc1a = pl.pallas_call(
    _tc1a_body,
    grid=(_GRID,),
    in_specs=[
        _row_spec(128),
        _full_spec(128, 128), _full_spec(1, 128),
        _full_spec(64, 128), _full_spec(1, 64),
        _full_spec(64, 64),
    ],
    out_specs=[_row_spec(GC), _row_spec(GC)],
    out_shape=[
        jax.ShapeDtypeStruct((N, GC), jnp.float32),
        jax.ShapeDtypeStruct((N, GC), jnp.float32),
    ],
    compiler_params=pltpu.CompilerParams(
        dimension_semantics=("arbitrary",)),
)

_tc1b = pl.pallas_call(
    _tc1b_body,
    grid=(_GRID,),
    in_specs=[_row_spec(GC), _row_spec(1)],
    out_specs=[_row_spec(GC)],
    out_shape=[jax.ShapeDtypeStruct((N, GC), jnp.float32)],
    compiler_params=pltpu.CompilerParams(
        dimension_semantics=("arbitrary",)),
)

_tc2 = pl.pallas_call(
    _tc2_body,
    grid=(_GRID,),
    in_specs=[
        _row_spec(GC), _row_spec(GC), _row_spec(GC), _row_spec(1),
        _full_spec(128, 64), _full_spec(128, 64),
        _full_spec(1, 128), _full_spec(1, 64),
    ],
    out_specs=[_row_spec(128)],
    out_shape=[jax.ShapeDtypeStruct((N, 128), jnp.float32)],
    compiler_params=pltpu.CompilerParams(
        dimension_semantics=("arbitrary",)),
)


@jax.jit
def kernel(feats, edges, batch, W1, b1, W2, b2, Wgc, bgc, Wc, bc):
    src = edges[0]
    dst = edges[1]
    zdeg = jnp.zeros((TPT,), jnp.float32)
    zrows = jnp.zeros((TPT, HGC), jnp.float32)

    deg_raw = _sc_deg(dst, zdeg)
    deg = jnp.concatenate([
        deg_raw[0].reshape(ACC)[:HALF],
        deg_raw[1].reshape(ACC)[:HALF],
    ]).reshape(N, 1)

    nf, xw = _tc1a(feats, W1, b1.reshape(1, -1), W2, b2.reshape(1, -1), Wgc)
    y, = _tc1b(xw, deg)

    out2a_raw, out2b_raw = _sc_scatter(src, dst, y[:, :HGC], y[:, HGC:],
                                       zrows)
    out2 = jnp.concatenate([
        jnp.concatenate([out2a_raw[i].reshape(ACC, HGC)[:HALF],
                         out2b_raw[i].reshape(ACC, HGC)[:HALF]], axis=1)
        for i in range(NC)
    ], axis=0)

    comb, = _tc2(nf, y, out2, deg, Wc[:, :GC], Wc[:, GC:],
                 bc.reshape(1, -1), bgc.reshape(1, -1))
    return (comb, edges, batch)
